# baseline (speedup 1.0000x reference)
import functools

import jax
import jax.numpy as jnp
from jax import lax
from jax.experimental import pallas as pl
from jax.experimental.pallas import tpu as pltpu
from jax.experimental.pallas import tpu_sc as plsc

N_ENTITY = 1000000
N_RELATION = 1000
DIM = 64
GAMMA = 12.0
EMB_RANGE = (GAMMA + 2.0) / DIM
PI = 3.141592653589793
BATCH = 4096
PHASE_K = PI / EMB_RANGE

NC, NS, L = 2, 16, 16
NW = NC * NS
BPW = BATCH // NW
NCHUNK = DIM // L
TROW_PW = 64

_SIN_C = (0.9999999403953552, -0.1666662096977234, 0.008332791738212109,
          -0.00019817629072349519, 2.708829470066121e-06,
          -2.0698067260127573e-08)
_COS_C = (1.0, -0.49999988079071045, 0.04166648909449577,
          -0.0013887803070247173, 2.47698826569831e-05,
          -2.7079025244347577e-07, 1.7245067596149966e-09)


def _horner(coeffs, t):
    acc = jnp.full((L,), coeffs[-1], jnp.float32)
    for c in coeffs[-2::-1]:
        acc = acc * t + jnp.float32(c)
    return acc


_SQRT12_C = (0.32566583156585693, 0.9148544073104858, -0.3164294958114624,
             0.086272232234478, -0.010351847857236862)


def _modulus16(re, im):
    a = jnp.abs(re)
    b = jnp.abs(im)
    hi = jnp.maximum(a, b)
    lo = jnp.minimum(a, b)
    ratio = lo / (hi + jnp.float32(1e-30))
    return hi * _horner(_SQRT12_C, jnp.float32(1.0) + ratio * ratio)


_mesh = plsc.VectorSubcoreMesh(core_axis_name="c", subcore_axis_name="s")


@functools.partial(
    pl.kernel,
    out_type=jax.ShapeDtypeStruct((BATCH,), jnp.float32),
    mesh=_mesh,
    compiler_params=pltpu.CompilerParams(needs_layout_passes=False,
                                         use_tc_tiling_on_sc=False),
    scratch_types=[
        pltpu.VMEM((BPW,), jnp.int32),
        pltpu.VMEM((BPW,), jnp.int32),
        pltpu.VMEM((BPW,), jnp.int32),
        pltpu.VMEM((BPW, 2 * DIM), jnp.float32),
        pltpu.VMEM((BPW, 2 * DIM), jnp.float32),
        pltpu.VMEM((BPW, 2 * DIM), jnp.float32),
        pltpu.VMEM((TROW_PW, DIM), jnp.float32),
        pltpu.VMEM((TROW_PW, 2 * DIM), jnp.float32),
        pltpu.VMEM_SHARED((N_RELATION, 2 * DIM), jnp.float32),
        pltpu.VMEM((BPW, L), jnp.float32),
        pltpu.VMEM((BPW,), jnp.float32),
        pltpu.SemaphoreType.DMA,
        pltpu.SemaphoreType.DMA,
        pltpu.SemaphoreType.DMA,
        pltpu.SemaphoreType.DMA,
        pltpu.SemaphoreType.DMA,
        pltpu.SemaphoreType.DMA,
        pltpu.SemaphoreType.DMA,
    ],
)
def _rotate_body(hrt_hbm, ent_hbm, rel_hbm, out_hbm,
                 hidx, ridx, tidx, head_v, tail_v, cs_v, trow_v, trig_v,
                 cs_shared, part_v, out_v,
                 sem0, sem1, sem2, sem3, sem_ih, sem_it, sem_ir):
    wid = lax.axis_index("s") * NC + lax.axis_index("c")
    base = wid * BPW
    sid = lax.axis_index("s")

    cp_ih = pltpu.async_copy(hrt_hbm.at[0, pl.ds(base, BPW)], hidx, sem_ih)
    cp_it = pltpu.async_copy(hrt_hbm.at[2, pl.ds(base, BPW)], tidx, sem_it)
    cp_ir = pltpu.async_copy(hrt_hbm.at[1, pl.ds(base, BPW)], ridx, sem_ir)

    sems = (sem0, sem1, sem2, sem3)
    NPH = len(sems)
    IPP = BPW // NPH
    copies = [[] for _ in range(NPH)]
    cp_ih.wait()
    for p in range(NPH):
        sl = pl.ds(p * IPP, IPP)
        copies[p].append(
            pltpu.async_copy(ent_hbm.at[hidx.at[sl]], head_v.at[sl], sems[p]))
    cp_it.wait()
    for p in range(NPH):
        sl = pl.ds(p * IPP, IPP)
        copies[p].append(
            pltpu.async_copy(ent_hbm.at[tidx.at[sl]], tail_v.at[sl], sems[p]))

    trow = jnp.minimum(sid * TROW_PW, N_RELATION - TROW_PW)
    pltpu.sync_copy(rel_hbm.at[pl.ds(trow, TROW_PW)], trow_v)

    def trig_row(i, carry):
        for j in range(NCHUNK):
            lo = j * L
            ph = trow_v[i, pl.ds(lo, L)] * jnp.float32(PHASE_K)
            t2 = ph * ph
            trig_v[i, pl.ds(lo, L)] = _horner(_COS_C, t2)
            trig_v[i, pl.ds(DIM + lo, L)] = ph * _horner(_SIN_C, t2)
        return carry

    lax.fori_loop(0, TROW_PW, trig_row, 0, unroll=1)
    pltpu.sync_copy(trig_v, cs_shared.at[pl.ds(trow, TROW_PW)])
    plsc.subcore_barrier()

    cp_ir.wait()
    for p in range(NPH):
        sl = pl.ds(p * IPP, IPP)
        copies[p].append(
            pltpu.async_copy(cs_shared.at[ridx.at[sl]], cs_v.at[sl], sems[p]))

    def item(i, carry):
        acc = jnp.zeros((L,), jnp.float32)
        for j in range(NCHUNK):
            lo = j * L
            cos_r = cs_v[i, pl.ds(lo, L)]
            sin_r = cs_v[i, pl.ds(DIM + lo, L)]
            re_t = tail_v[i, pl.ds(lo, L)]
            im_t = tail_v[i, pl.ds(DIM + lo, L)]
            re_h = head_v[i, pl.ds(lo, L)]
            im_h = head_v[i, pl.ds(DIM + lo, L)]
            re_s = cos_r * re_t + sin_r * im_t - re_h
            im_s = cos_r * im_t - sin_r * re_t - im_h
            acc = acc + _modulus16(re_s, im_s)
        part_v[i, pl.ds(0, L)] = acc
        return carry

    for p in range(NPH):
        for cp in copies[p]:
            cp.wait()
        lax.fori_loop(p * IPP, (p + 1) * IPP, item, 0, unroll=1)

    iota = lax.iota(jnp.int32, L)

    def group(g, carry):
        items = iota + g * L

        def dim(d, tot):
            return tot + plsc.load_gather(
                part_v, [items, jnp.broadcast_to(d, (L,))])

        tot = lax.fori_loop(0, L, dim, jnp.zeros((L,), jnp.float32))
        out_v[pl.ds(g * L, L)] = -tot
        return carry

    lax.fori_loop(0, BPW // L, group, 0)

    pltpu.sync_copy(out_v, out_hbm.at[pl.ds(base, BPW)])


def kernel(h, r, t, entity_embedding, relation_embedding):
    hrt = jnp.stack([h, r, t]).astype(jnp.int32)
    flat = _rotate_body(hrt, entity_embedding, relation_embedding)
    return flat.reshape(BATCH, 1)


# speedup vs baseline: 1.1379x; 1.1379x over previous
import functools

import jax
import jax.numpy as jnp
from jax import lax
from jax.experimental import pallas as pl
from jax.experimental.pallas import tpu as pltpu
from jax.experimental.pallas import tpu_sc as plsc

N_ENTITY = 1000000
N_RELATION = 1000
DIM = 64
GAMMA = 12.0
EMB_RANGE = (GAMMA + 2.0) / DIM
PI = 3.141592653589793
BATCH = 4096
PHASE_K = PI / EMB_RANGE

NC, NS, L = 2, 16, 16
NW = NC * NS
BPW = BATCH // NW
NCHUNK = DIM // L

_SIN_C = (0.9999999403953552, -0.1666662096977234, 0.008332791738212109,
          -0.00019817629072349519, 2.708829470066121e-06,
          -2.0698067260127573e-08)
_COS_C = (1.0, -0.49999988079071045, 0.04166648909449577,
          -0.0013887803070247173, 2.47698826569831e-05,
          -2.7079025244347577e-07, 1.7245067596149966e-09)


def _horner(coeffs, t):
    acc = jnp.full((L,), coeffs[-1], jnp.float32)
    for c in coeffs[-2::-1]:
        acc = acc * t + jnp.float32(c)
    return acc


_SQRT12_C = (0.32566583156585693, 0.9148544073104858, -0.3164294958114624,
             0.086272232234478, -0.010351847857236862)


def _modulus16(re, im):
    a = jnp.abs(re)
    b = jnp.abs(im)
    hi = jnp.maximum(a, b)
    lo = jnp.minimum(a, b)
    ratio = lo / (hi + jnp.float32(1e-30))
    return hi * _horner(_SQRT12_C, jnp.float32(1.0) + ratio * ratio)


_mesh = plsc.VectorSubcoreMesh(core_axis_name="c", subcore_axis_name="s")


@functools.partial(
    pl.kernel,
    out_type=jax.ShapeDtypeStruct((BATCH,), jnp.float32),
    mesh=_mesh,
    compiler_params=pltpu.CompilerParams(needs_layout_passes=False,
                                         use_tc_tiling_on_sc=False),
    scratch_types=[
        pltpu.VMEM((BPW,), jnp.int32),
        pltpu.VMEM((BPW,), jnp.int32),
        pltpu.VMEM((BPW,), jnp.int32),
        pltpu.VMEM((BPW, 2 * DIM), jnp.float32),
        pltpu.VMEM((BPW, 2 * DIM), jnp.float32),
        pltpu.VMEM((BPW, DIM), jnp.float32),
        pltpu.VMEM((BPW, L), jnp.float32),
        pltpu.VMEM((BPW,), jnp.float32),
        pltpu.SemaphoreType.DMA,
        pltpu.SemaphoreType.DMA,
        pltpu.SemaphoreType.DMA,
        pltpu.SemaphoreType.DMA,
        pltpu.SemaphoreType.DMA,
        pltpu.SemaphoreType.DMA,
        pltpu.SemaphoreType.DMA,
    ],
)
def _rotate_body(h_hbm, r_hbm, t_hbm, ent_hbm, rel_hbm, out_hbm,
                 hidx, ridx, tidx, head_v, tail_v, rel_v, part_v, out_v,
                 sem0, sem1, sem2, sem3, sem_ih, sem_it, sem_ir):
    wid = lax.axis_index("s") * NC + lax.axis_index("c")
    base = wid * BPW

    cp_ih = pltpu.async_copy(h_hbm.at[pl.ds(base, BPW)], hidx, sem_ih)
    cp_it = pltpu.async_copy(t_hbm.at[pl.ds(base, BPW)], tidx, sem_it)
    cp_ir = pltpu.async_copy(r_hbm.at[pl.ds(base, BPW)], ridx, sem_ir)

    sems = (sem0, sem1, sem2, sem3)
    NPH = len(sems)
    IPP = BPW // NPH
    copies = [[] for _ in range(NPH)]
    cp_ih.wait()
    for p in range(NPH):
        sl = pl.ds(p * IPP, IPP)
        copies[p].append(
            pltpu.async_copy(ent_hbm.at[hidx.at[sl]], head_v.at[sl], sems[p]))
    cp_it.wait()
    for p in range(NPH):
        sl = pl.ds(p * IPP, IPP)
        copies[p].append(
            pltpu.async_copy(ent_hbm.at[tidx.at[sl]], tail_v.at[sl], sems[p]))
    cp_ir.wait()
    for p in range(NPH):
        sl = pl.ds(p * IPP, IPP)
        copies[p].append(
            pltpu.async_copy(rel_hbm.at[ridx.at[sl]], rel_v.at[sl], sems[p]))

    def item(i, carry):
        acc = jnp.zeros((L,), jnp.float32)
        for j in range(NCHUNK):
            lo = j * L
            ph = rel_v[i, pl.ds(lo, L)] * jnp.float32(PHASE_K)
            t2 = ph * ph
            cos_r = _horner(_COS_C, t2)
            sin_r = ph * _horner(_SIN_C, t2)
            re_t = tail_v[i, pl.ds(lo, L)]
            im_t = tail_v[i, pl.ds(DIM + lo, L)]
            re_h = head_v[i, pl.ds(lo, L)]
            im_h = head_v[i, pl.ds(DIM + lo, L)]
            re_s = cos_r * re_t + sin_r * im_t - re_h
            im_s = cos_r * im_t - sin_r * re_t - im_h
            acc = acc + _modulus16(re_s, im_s)
        part_v[i, pl.ds(0, L)] = acc
        return carry

    for p in range(NPH):
        for cp in copies[p]:
            cp.wait()
        lax.fori_loop(p * IPP, (p + 1) * IPP, item, 0, unroll=1)

    iota = lax.iota(jnp.int32, L)

    def group(g, carry):
        items = iota + g * L

        def dim(d, tot):
            return tot + plsc.load_gather(
                part_v, [items, jnp.broadcast_to(d, (L,))])

        tot = lax.fori_loop(0, L, dim, jnp.zeros((L,), jnp.float32))
        out_v[pl.ds(g * L, L)] = -tot
        return carry

    lax.fori_loop(0, BPW // L, group, 0)

    pltpu.sync_copy(out_v, out_hbm.at[pl.ds(base, BPW)])


def kernel(h, r, t, entity_embedding, relation_embedding):
    if h.dtype != jnp.int32:
        h, r, t = (x.astype(jnp.int32) for x in (h, r, t))
    flat = _rotate_body(h, r, t, entity_embedding, relation_embedding)
    return flat.reshape(BATCH, 1)
